# Initial kernel scaffold; baseline (speedup 1.0000x reference)
import functools

import jax
import jax.numpy as jnp
from jax import lax
from jax.experimental import pallas as pl
from jax.experimental.pallas import tpu as pltpu
from jax.experimental.pallas import tpu_sc as plsc

N = 883
NP = 896
B, T = 4, 12
R = B * T
E = 14128
NTILES = 32
ROWS_PER_TILE = NP // NTILES
BLK_WORDS = ROWS_PER_TILE * NP
LANES = 16


def _sc_lane_gather(v, idx):
    return lax.gather(
        v, idx.reshape(LANES, 1),
        lax.GatherDimensionNumbers(
            offset_dims=(), collapsed_slice_dims=(0,), start_index_map=(0,)),
        (1,), mode=lax.GatherScatterMode.PROMISE_IN_BOUNDS)


def _build_adj_kernel(src_hbm, dst_hbm, out_hbm, src_v, dst_v, acc):
    wid = lax.axis_index("c") * 16 + lax.axis_index("s")
    base_row = wid * ROWS_PER_TILE
    iota = lax.broadcasted_iota(jnp.int32, (LANES,), 0)
    zeros16 = jnp.zeros((LANES,), jnp.float32)
    ones16 = jnp.ones((LANES,), jnp.float32)

    pltpu.sync_copy(src_hbm, src_v)
    pltpu.sync_copy(dst_hbm, dst_v)

    def _zero(j, carry):
        acc[pl.ds(j * LANES, LANES)] = zeros16
        return carry
    lax.fori_loop(0, BLK_WORDS // LANES, _zero, 0)

    big = jnp.int32(0x40000000)

    def _edges(g, carry):
        s = src_v[pl.ds(g * LANES, LANES)]
        d = dst_v[pl.ds(g * LANES, LANES)]
        rel = d - base_row
        inr = (rel >= 0) & (rel < ROWS_PER_TILE)
        idx = rel * NP + s
        key = jnp.where(inr, idx, big + iota)
        skey, _ = plsc.sort_key_val(key, key)
        prev = _sc_lane_gather(skey, jnp.maximum(iota - 1, 0))
        nxt = _sc_lane_gather(skey, jnp.minimum(iota + 1, LANES - 1))
        is_first = (skey != prev) | (iota == 0)
        is_last = (skey != nxt) | (iota == LANES - 1)
        first_pos = plsc.cummax(jnp.where(is_first, iota, -1))
        cnt = (iota - first_pos + 1).astype(jnp.float32)
        wmask = is_last & (skey < big)
        widx = jnp.where(wmask, skey, 0)
        plsc.addupdate_scatter(acc, [widx], cnt, wmask)
        return carry
    lax.fori_loop(0, E // LANES, _edges, 0)

    diag0 = iota * (NP + 1) + base_row
    plsc.addupdate_scatter(acc, [diag0], ones16,
                           iota < jnp.int32(ROWS_PER_TILE))
    diag1 = (iota + LANES) * (NP + 1) + base_row
    plsc.addupdate_scatter(acc, [jnp.where(iota < ROWS_PER_TILE - LANES,
                                           diag1, 0)],
                           ones16, iota < jnp.int32(ROWS_PER_TILE - LANES))

    pltpu.sync_copy(acc, out_hbm.at[pl.ds(wid * BLK_WORDS, BLK_WORDS)])


def _build_adj(src, dst):
    k = functools.partial(
        pl.kernel,
        out_type=jax.ShapeDtypeStruct((NP * NP,), jnp.float32),
        mesh=plsc.VectorSubcoreMesh(core_axis_name="c", subcore_axis_name="s"),
        scratch_types=[
            pltpu.VMEM((E,), jnp.int32),
            pltpu.VMEM((E,), jnp.int32),
            pltpu.VMEM((BLK_WORDS,), jnp.float32),
        ],
    )(_build_adj_kernel)
    return k(src, dst)


def _nt(a, b):
    return lax.dot_general(a, b, (((1,), (1,)), ((), ())),
                           preferred_element_type=jnp.float32)


def _gat_layer(h_in, valid, aeff, W, asv, adv, bias, heads):
    hm = jnp.dot(h_in, W, preferred_element_type=jnp.float32)
    hs = hm * asv
    hd = hm * adv
    fo = 64 // heads
    if heads > 1:
        gt = (lax.broadcasted_iota(jnp.int32, (heads, 64), 1) // fo
              == lax.broadcasted_iota(jnp.int32, (heads, 64), 0))
        gt = gt.astype(jnp.float32)
        al_st = _nt(gt, hs)
        al_d = _nt(hd, gt)
    else:
        ones1 = jnp.ones((1, 64), jnp.float32)
        al_st = _nt(ones1, hs)
        al_d = _nt(hd, ones1)
    outs = []
    for hh in range(heads):
        e = al_d[:, hh:hh + 1] + al_st[hh:hh + 1, :]
        e = jnp.where(e > 0.0, e, 0.2 * e)
        em = jnp.where(valid, e, -1e30)
        m = jnp.max(em, axis=1, keepdims=True)
        p = jnp.exp(em - m) * aeff
        den = jnp.sum(p, axis=1, keepdims=True)
        ph = jnp.dot(p, hm[:, hh * fo:(hh + 1) * fo],
                     preferred_element_type=jnp.float32)
        outs.append(ph / (den + 1e-16))
    out = outs[0] if heads == 1 else jnp.concatenate(outs, axis=1)
    return out + bias


def _gat3_body(A_ref, x_ref, W0_ref, as0_ref, ad0_ref, b0_ref,
               W1_ref, as1_ref, ad1_ref, b1_ref,
               W2_ref, as2_ref, ad2_ref, b2_ref, out_ref):
    aeff = A_ref[...]
    valid = aeff > 0.0
    x = x_ref[0]
    h = _gat_layer(x, valid, aeff, W0_ref[...], as0_ref[...], ad0_ref[...],
                   b0_ref[...], 8)
    h = jnp.where(h > 0.0, h, jnp.exp(h) - 1.0)
    h = _gat_layer(h, valid, aeff, W1_ref[...], as1_ref[...], ad1_ref[...],
                   b1_ref[...], 8)
    h = jnp.where(h > 0.0, h, jnp.exp(h) - 1.0)
    h = _gat_layer(h, valid, aeff, W2_ref[...], as2_ref[...], ad2_ref[...],
                   b2_ref[...], 1)
    out_ref[0] = h


def _gat3(A, xp, W0, as0v, ad0v, b0v, W1, as1v, ad1v, b1v,
          W2, as2v, ad2v, b2v):
    full = lambda shape: pl.BlockSpec(shape, lambda r: (0,) * len(shape))
    return pl.pallas_call(
        _gat3_body,
        grid=(R,),
        in_specs=[
            full((NP, NP)),
            pl.BlockSpec((1, NP, 3), lambda r: (r, 0, 0)),
            full((3, 64)), full((1, 64)), full((1, 64)), full((1, 64)),
            full((64, 64)), full((1, 64)), full((1, 64)), full((1, 64)),
            full((64, 64)), full((1, 64)), full((1, 64)), full((1, 64)),
        ],
        out_specs=pl.BlockSpec((1, NP, 64), lambda r: (r, 0, 0)),
        out_shape=jax.ShapeDtypeStruct((R, NP, 64), jnp.float32),
    )(A, xp, W0, as0v, ad0v, b0v, W1, as1v, ad1v, b1v, W2, as2v, ad2v, b2v)


NBLK = 128
NJ = NP // NBLK


def _lstm(ys, Wih, Whh, bsum):
    y_all = jnp.concatenate(ys, axis=0)
    gx = _nt(y_all, Wih) + bsum
    h = jnp.zeros((NBLK, 64), jnp.float32)
    c = jnp.zeros((NBLK, 64), jnp.float32)
    outs = []
    for t in range(T):
        g = gx[t * NBLK:(t + 1) * NBLK, :] + _nt(h, Whh)
        i = jax.nn.sigmoid(g[:, 0:64])
        f = jax.nn.sigmoid(g[:, 64:128])
        gg = jnp.tanh(g[:, 128:192])
        o = jax.nn.sigmoid(g[:, 192:256])
        c = f * c + i * gg
        h = o * jnp.tanh(c)
        outs.append(h)
    return outs, h


def _temporal_body(tf_ref, cwT_ref, cb_ref, Wih0_ref, Whh0_ref, bL0_ref,
                   Wih1_ref, Whh1_ref, bL1_ref, ow1_ref, ob1_ref,
                   ow2_ref, ob2_ref, out_ref):
    xb = tf_ref[0]
    xf = xb.reshape(T * NBLK, 64)
    y0 = _nt(xf, cwT_ref[0])
    y1 = _nt(xf, cwT_ref[1])
    y2 = _nt(xf, cwT_ref[2])
    cb = cb_ref[...]
    ys = []
    for t in range(T):
        y = y1[t * NBLK:(t + 1) * NBLK, :] + cb
        if t > 0:
            y = y + y0[(t - 1) * NBLK:t * NBLK, :]
        if t < T - 1:
            y = y + y2[(t + 1) * NBLK:(t + 2) * NBLK, :]
        ys.append(y)
    ys1, _ = _lstm(ys, Wih0_ref[...], Whh0_ref[...], bL0_ref[...])
    _, h2 = _lstm(ys1, Wih1_ref[...], Whh1_ref[...], bL1_ref[...])
    hid = jnp.maximum(_nt(h2, ow1_ref[...]) + ob1_ref[...], 0.0)
    out_ref[0] = _nt(hid, ow2_ref[...]) + ob2_ref[...]


def _temporal(tf4, cwT, cbv, Wih0, Whh0, bL0, Wih1, Whh1, bL1,
              ow1, ob1v, ow2, ob2v):
    full = lambda shape: pl.BlockSpec(shape, lambda b, j: (0,) * len(shape))
    return pl.pallas_call(
        _temporal_body,
        grid=(B, NJ),
        in_specs=[
            pl.BlockSpec((1, T, NBLK, 64), lambda b, j: (b, 0, j, 0)),
            full((3, 64, 64)), full((1, 64)),
            full((256, 64)), full((256, 64)), full((1, 256)),
            full((256, 64)), full((256, 64)), full((1, 256)),
            full((32, 64)), full((1, 32)), full((24, 32)), full((1, 24)),
        ],
        out_specs=pl.BlockSpec((1, NBLK, 24), lambda b, j: (b, j, 0)),
        out_shape=jax.ShapeDtypeStruct((B, NP, 24), jnp.float32),
    )(tf4, cwT, cbv, Wih0, Whh0, bL0, Wih1, Whh1, bL1, ow1, ob1v, ow2, ob2v)


def kernel(x, W0, as0, ad0, b0, W1, as1, ad1, b1, W2, as2, ad2, b2, cw, cb,
           Wih0, Whh0, bih0, bhh0, Wih1, Whh1, bih1, bhh1, ow1, ob1, ow2,
           ob2, edge_index):
    src = edge_index[0].astype(jnp.int32)
    dst = edge_index[1].astype(jnp.int32)
    A = _build_adj(src, dst).reshape(NP, NP)

    xp = jnp.pad(x, ((0, 0), (0, 0), (0, NP - N), (0, 0)))
    xp = xp.reshape(R, NP, 3)
    tf = _gat3(A, xp,
               W0, as0.reshape(1, 64), ad0.reshape(1, 64), b0.reshape(1, 64),
               W1, as1.reshape(1, 64), ad1.reshape(1, 64), b1.reshape(1, 64),
               W2, as2.reshape(1, 64), ad2.reshape(1, 64), b2.reshape(1, 64))
    tf4 = tf.reshape(B, T, NP, 64)

    cwT = jnp.transpose(cw, (2, 0, 1))
    y = _temporal(tf4, cwT, cb.reshape(1, 64),
                  Wih0, Whh0, (bih0 + bhh0).reshape(1, 256),
                  Wih1, Whh1, (bih1 + bhh1).reshape(1, 256),
                  ow1, ob1.reshape(1, 32), ow2, ob2.reshape(1, 24))
    return y[:, :N, :]


# speedup vs baseline: 144.5414x; 144.5414x over previous
import functools

import jax
import jax.numpy as jnp
from jax import lax
from jax.experimental import pallas as pl
from jax.experimental.pallas import tpu as pltpu
from jax.experimental.pallas import tpu_sc as plsc

N = 883
NP = 896
B, T = 4, 12
R = B * T
E = 14128
NTILES = 32
ROWS_PER_TILE = NP // NTILES
BLK_WORDS = ROWS_PER_TILE * NP
LANES = 16


def _sc_lane_gather(v, idx):
    return lax.gather(
        v, idx.reshape(LANES, 1),
        lax.GatherDimensionNumbers(
            offset_dims=(), collapsed_slice_dims=(0,), start_index_map=(0,)),
        (1,), mode=lax.GatherScatterMode.PROMISE_IN_BOUNDS)


def _build_adj_kernel(src_hbm, dst_hbm, out_hbm, src_v, dst_v, acc):
    wid = lax.axis_index("c") * 16 + lax.axis_index("s")
    base_row = wid * ROWS_PER_TILE
    iota = lax.broadcasted_iota(jnp.int32, (LANES,), 0)
    zeros16 = jnp.zeros((LANES,), jnp.float32)
    ones16 = jnp.ones((LANES,), jnp.float32)

    pltpu.sync_copy(src_hbm, src_v)
    pltpu.sync_copy(dst_hbm, dst_v)

    def _zero(j, carry):
        acc[pl.ds(j * LANES, LANES)] = zeros16
        return carry
    lax.fori_loop(0, BLK_WORDS // LANES, _zero, 0)

    big = jnp.int32(0x40000000)

    def _edges(g, carry):
        s = src_v[pl.ds(g * LANES, LANES)]
        d = dst_v[pl.ds(g * LANES, LANES)]
        rel = d - base_row
        inr = (rel >= 0) & (rel < ROWS_PER_TILE)
        idx = rel * NP + s
        key = jnp.where(inr, idx, big + iota)
        cnt, last = plsc.scan_count(key, mask=inr)
        wmask = last & inr
        widx = jnp.where(wmask, key, 0)
        plsc.addupdate_scatter(acc, [widx], cnt.astype(jnp.float32),
                               mask=wmask)
        return carry
    lax.fori_loop(0, E // LANES, _edges, 0)

    diag0 = iota * (NP + 1) + base_row
    plsc.addupdate_scatter(acc, [diag0], ones16,
                           mask=iota < jnp.int32(ROWS_PER_TILE))
    diag1 = (iota + LANES) * (NP + 1) + base_row
    plsc.addupdate_scatter(acc, [jnp.where(iota < ROWS_PER_TILE - LANES,
                                           diag1, 0)],
                           ones16,
                           mask=iota < jnp.int32(ROWS_PER_TILE - LANES))

    pltpu.sync_copy(acc, out_hbm.at[pl.ds(wid * BLK_WORDS, BLK_WORDS)])


def _build_adj(src, dst):
    k = functools.partial(
        pl.kernel,
        out_type=jax.ShapeDtypeStruct((NP * NP,), jnp.float32),
        mesh=plsc.VectorSubcoreMesh(core_axis_name="c", subcore_axis_name="s"),
        compiler_params=pltpu.CompilerParams(needs_layout_passes=False),
        scratch_types=[
            pltpu.VMEM((E,), jnp.int32),
            pltpu.VMEM((E,), jnp.int32),
            pltpu.VMEM((BLK_WORDS,), jnp.float32),
        ],
    )(_build_adj_kernel)
    return k(src, dst)


def _nt(a, b):
    return lax.dot_general(a, b, (((1,), (1,)), ((), ())),
                           preferred_element_type=jnp.float32)


def _gat_layer(h_in, valid, aeff, W, asv, adv, bias, heads):
    hm = jnp.dot(h_in, W, preferred_element_type=jnp.float32)
    hs = hm * asv
    hd = hm * adv
    fo = 64 // heads
    if heads > 1:
        gt = (lax.broadcasted_iota(jnp.int32, (heads, 64), 1) // fo
              == lax.broadcasted_iota(jnp.int32, (heads, 64), 0))
        gt = gt.astype(jnp.float32)
        al_st = _nt(gt, hs)
        al_d = _nt(hd, gt)
    else:
        ones1 = jnp.ones((1, 64), jnp.float32)
        al_st = _nt(ones1, hs)
        al_d = _nt(hd, ones1)
    outs = []
    for hh in range(heads):
        e = al_d[:, hh:hh + 1] + al_st[hh:hh + 1, :]
        e = jnp.where(e > 0.0, e, 0.2 * e)
        em = jnp.where(valid, e, -1e30)
        m = jnp.max(em, axis=1, keepdims=True)
        p = jnp.exp(em - m) * aeff
        den = jnp.sum(p, axis=1, keepdims=True)
        ph = jnp.dot(p, hm[:, hh * fo:(hh + 1) * fo],
                     preferred_element_type=jnp.float32)
        outs.append(ph / (den + 1e-16))
    out = outs[0] if heads == 1 else jnp.concatenate(outs, axis=1)
    return out + bias


def _gat3_body(A_ref, x_ref, W0_ref, as0_ref, ad0_ref, b0_ref,
               W1_ref, as1_ref, ad1_ref, b1_ref,
               W2_ref, as2_ref, ad2_ref, b2_ref, out_ref):
    aeff = A_ref[...]
    valid = aeff > 0.0
    x = x_ref[0]
    h = _gat_layer(x, valid, aeff, W0_ref[...], as0_ref[...], ad0_ref[...],
                   b0_ref[...], 8)
    h = jnp.where(h > 0.0, h, jnp.exp(h) - 1.0)
    h = _gat_layer(h, valid, aeff, W1_ref[...], as1_ref[...], ad1_ref[...],
                   b1_ref[...], 8)
    h = jnp.where(h > 0.0, h, jnp.exp(h) - 1.0)
    h = _gat_layer(h, valid, aeff, W2_ref[...], as2_ref[...], ad2_ref[...],
                   b2_ref[...], 1)
    out_ref[0] = h


def _gat3(A, xp, W0, as0v, ad0v, b0v, W1, as1v, ad1v, b1v,
          W2, as2v, ad2v, b2v):
    full = lambda shape: pl.BlockSpec(shape, lambda r: (0,) * len(shape))
    return pl.pallas_call(
        _gat3_body,
        grid=(R,),
        in_specs=[
            full((NP, NP)),
            pl.BlockSpec((1, NP, 3), lambda r: (r, 0, 0)),
            full((3, 64)), full((1, 64)), full((1, 64)), full((1, 64)),
            full((64, 64)), full((1, 64)), full((1, 64)), full((1, 64)),
            full((64, 64)), full((1, 64)), full((1, 64)), full((1, 64)),
        ],
        out_specs=pl.BlockSpec((1, NP, 64), lambda r: (r, 0, 0)),
        out_shape=jax.ShapeDtypeStruct((R, NP, 64), jnp.float32),
    )(A, xp, W0, as0v, ad0v, b0v, W1, as1v, ad1v, b1v, W2, as2v, ad2v, b2v)


NBLK = 128
NJ = NP // NBLK


def _lstm(ys, Wih, Whh, bsum):
    y_all = jnp.concatenate(ys, axis=0)
    gx = _nt(y_all, Wih) + bsum
    h = jnp.zeros((NBLK, 64), jnp.float32)
    c = jnp.zeros((NBLK, 64), jnp.float32)
    outs = []
    for t in range(T):
        g = gx[t * NBLK:(t + 1) * NBLK, :] + _nt(h, Whh)
        i = jax.nn.sigmoid(g[:, 0:64])
        f = jax.nn.sigmoid(g[:, 64:128])
        gg = jnp.tanh(g[:, 128:192])
        o = jax.nn.sigmoid(g[:, 192:256])
        c = f * c + i * gg
        h = o * jnp.tanh(c)
        outs.append(h)
    return outs, h


def _temporal_body(tf_ref, cwT_ref, cb_ref, Wih0_ref, Whh0_ref, bL0_ref,
                   Wih1_ref, Whh1_ref, bL1_ref, ow1_ref, ob1_ref,
                   ow2_ref, ob2_ref, out_ref):
    xb = tf_ref[0]
    xf = xb.reshape(T * NBLK, 64)
    y0 = _nt(xf, cwT_ref[0])
    y1 = _nt(xf, cwT_ref[1])
    y2 = _nt(xf, cwT_ref[2])
    cb = cb_ref[...]
    ys = []
    for t in range(T):
        y = y1[t * NBLK:(t + 1) * NBLK, :] + cb
        if t > 0:
            y = y + y0[(t - 1) * NBLK:t * NBLK, :]
        if t < T - 1:
            y = y + y2[(t + 1) * NBLK:(t + 2) * NBLK, :]
        ys.append(y)
    ys1, _ = _lstm(ys, Wih0_ref[...], Whh0_ref[...], bL0_ref[...])
    _, h2 = _lstm(ys1, Wih1_ref[...], Whh1_ref[...], bL1_ref[...])
    hid = jnp.maximum(_nt(h2, ow1_ref[...]) + ob1_ref[...], 0.0)
    out_ref[0] = _nt(hid, ow2_ref[...]) + ob2_ref[...]


def _temporal(tf4, cwT, cbv, Wih0, Whh0, bL0, Wih1, Whh1, bL1,
              ow1, ob1v, ow2, ob2v):
    full = lambda shape: pl.BlockSpec(shape, lambda b, j: (0,) * len(shape))
    return pl.pallas_call(
        _temporal_body,
        grid=(B, NJ),
        in_specs=[
            pl.BlockSpec((1, T, NBLK, 64), lambda b, j: (b, 0, j, 0)),
            full((3, 64, 64)), full((1, 64)),
            full((256, 64)), full((256, 64)), full((1, 256)),
            full((256, 64)), full((256, 64)), full((1, 256)),
            full((32, 64)), full((1, 32)), full((24, 32)), full((1, 24)),
        ],
        out_specs=pl.BlockSpec((1, NBLK, 24), lambda b, j: (b, j, 0)),
        out_shape=jax.ShapeDtypeStruct((B, NP, 24), jnp.float32),
    )(tf4, cwT, cbv, Wih0, Whh0, bL0, Wih1, Whh1, bL1, ow1, ob1v, ow2, ob2v)


def kernel(x, W0, as0, ad0, b0, W1, as1, ad1, b1, W2, as2, ad2, b2, cw, cb,
           Wih0, Whh0, bih0, bhh0, Wih1, Whh1, bih1, bhh1, ow1, ob1, ow2,
           ob2, edge_index):
    src = edge_index[0].astype(jnp.int32)
    dst = edge_index[1].astype(jnp.int32)
    A = _build_adj(src, dst).reshape(NP, NP)

    xp = jnp.pad(x, ((0, 0), (0, 0), (0, NP - N), (0, 0)))
    xp = xp.reshape(R, NP, 3)
    tf = _gat3(A, xp,
               W0, as0.reshape(1, 64), ad0.reshape(1, 64), b0.reshape(1, 64),
               W1, as1.reshape(1, 64), ad1.reshape(1, 64), b1.reshape(1, 64),
               W2, as2.reshape(1, 64), ad2.reshape(1, 64), b2.reshape(1, 64))
    tf4 = tf.reshape(B, T, NP, 64)

    cwT = jnp.transpose(cw, (2, 0, 1))
    y = _temporal(tf4, cwT, cb.reshape(1, 64),
                  Wih0, Whh0, (bih0 + bhh0).reshape(1, 256),
                  Wih1, Whh1, (bih1 + bhh1).reshape(1, 256),
                  ow1, ob1.reshape(1, 32), ow2, ob2.reshape(1, 24))
    return y[:, :N, :]


# speedup vs baseline: 274.3029x; 1.8977x over previous
import functools

import jax
import jax.numpy as jnp
from jax import lax
from jax.experimental import pallas as pl
from jax.experimental.pallas import tpu as pltpu
from jax.experimental.pallas import tpu_sc as plsc

N = 883
NP = 896
B, T = 4, 12
R = B * T
E = 14128
NTILES = 32
ROWS_PER_TILE = NP // NTILES
BLK_WORDS = ROWS_PER_TILE * NP
LANES = 16


def _sc_lane_gather(v, idx):
    return lax.gather(
        v, idx.reshape(LANES, 1),
        lax.GatherDimensionNumbers(
            offset_dims=(), collapsed_slice_dims=(0,), start_index_map=(0,)),
        (1,), mode=lax.GatherScatterMode.PROMISE_IN_BOUNDS)


def _build_adj_kernel(src_hbm, dst_hbm, out_hbm, src_v, dst_v, acc):
    wid = lax.axis_index("c") * 16 + lax.axis_index("s")
    base_row = wid * ROWS_PER_TILE
    iota = lax.broadcasted_iota(jnp.int32, (LANES,), 0)
    zeros16 = jnp.zeros((LANES,), jnp.float32)
    ones16 = jnp.ones((LANES,), jnp.float32)

    pltpu.sync_copy(src_hbm, src_v)
    pltpu.sync_copy(dst_hbm, dst_v)

    def _zero(j, carry):
        acc[pl.ds(j * LANES, LANES)] = zeros16
        return carry
    lax.fori_loop(0, BLK_WORDS // LANES, _zero, 0)

    big = jnp.int32(0x40000000)

    def _edges(g, carry):
        s = src_v[pl.ds(g * LANES, LANES)]
        d = dst_v[pl.ds(g * LANES, LANES)]
        rel = d - base_row
        inr = (rel >= 0) & (rel < ROWS_PER_TILE)
        idx = rel * NP + s
        key = jnp.where(inr, idx, big + iota)
        cnt, last = plsc.scan_count(key, mask=inr)
        wmask = last & inr
        widx = jnp.where(wmask, key, 0)
        plsc.addupdate_scatter(acc, [widx], cnt.astype(jnp.float32),
                               mask=wmask)
        return carry
    lax.fori_loop(0, E // LANES, _edges, 0)

    diag0 = iota * (NP + 1) + base_row
    plsc.addupdate_scatter(acc, [diag0], ones16,
                           mask=iota < jnp.int32(ROWS_PER_TILE))
    diag1 = (iota + LANES) * (NP + 1) + base_row
    plsc.addupdate_scatter(acc, [jnp.where(iota < ROWS_PER_TILE - LANES,
                                           diag1, 0)],
                           ones16,
                           mask=iota < jnp.int32(ROWS_PER_TILE - LANES))

    pltpu.sync_copy(acc, out_hbm.at[pl.ds(wid * BLK_WORDS, BLK_WORDS)])


def _build_adj(src, dst):
    k = functools.partial(
        pl.kernel,
        out_type=jax.ShapeDtypeStruct((NP * NP,), jnp.float32),
        mesh=plsc.VectorSubcoreMesh(core_axis_name="c", subcore_axis_name="s"),
        compiler_params=pltpu.CompilerParams(needs_layout_passes=False),
        scratch_types=[
            pltpu.VMEM((E,), jnp.int32),
            pltpu.VMEM((E,), jnp.int32),
            pltpu.VMEM((BLK_WORDS,), jnp.float32),
        ],
    )(_build_adj_kernel)
    return k(src, dst)


def _nt(a, b):
    return lax.dot_general(a, b, (((1,), (1,)), ((), ())),
                           preferred_element_type=jnp.float32)


def _gat_layer(h_in, aeff, W, asv, adv, bias, heads):
    hm = jnp.dot(h_in, W, preferred_element_type=jnp.float32)
    hs = hm * asv
    hd = hm * adv
    fo = 64 // heads
    if heads > 1:
        gt = (lax.broadcasted_iota(jnp.int32, (heads, 64), 1) // fo
              == lax.broadcasted_iota(jnp.int32, (heads, 64), 0))
        gt = gt.astype(jnp.float32)
        al_st = _nt(gt, hs)
        al_d = _nt(hd, gt)
    else:
        ones8 = jnp.ones((8, 64), jnp.float32)
        al_st = _nt(ones8, hs)[0:1, :]
        al_d = _nt(hd, ones8)[:, 0:1]
    ones_col = jnp.ones((NP, 1), jnp.float32)
    outs = []
    for hh in range(heads):
        avs = al_st[hh:hh + 1, :]
        amax = jnp.max(avs, axis=1, keepdims=True)
        zd = al_d[:, hh:hh + 1] + amax
        mp = jnp.maximum(zd, 0.2 * zd)
        u1 = jnp.exp(zd - mp)
        u2 = jnp.exp(0.2 * zd - mp)
        vs = avs - amax
        v1 = jnp.exp(vs)
        v2 = jnp.exp(0.2 * vs)
        p = jnp.maximum(u1 * v1, u2 * v2) * aeff
        hcat = jnp.concatenate(
            [hm[:, hh * fo:(hh + 1) * fo], ones_col], axis=1)
        phd = jnp.dot(p, hcat, preferred_element_type=jnp.float32)
        outs.append(phd[:, :fo] / (phd[:, fo:fo + 1] + 1e-16))
    out = outs[0] if heads == 1 else jnp.concatenate(outs, axis=1)
    return out + bias


def _gat3_body(A_ref, x_ref, W0_ref, as0_ref, ad0_ref, b0_ref,
               W1_ref, as1_ref, ad1_ref, b1_ref,
               W2_ref, as2_ref, ad2_ref, b2_ref, out_ref):
    aeff = A_ref[...]
    x = x_ref[0]
    h = _gat_layer(x, aeff, W0_ref[...], as0_ref[...], ad0_ref[...],
                   b0_ref[...], 8)
    h = jnp.where(h > 0.0, h, jnp.exp(h) - 1.0)
    h = _gat_layer(h, aeff, W1_ref[...], as1_ref[...], ad1_ref[...],
                   b1_ref[...], 8)
    h = jnp.where(h > 0.0, h, jnp.exp(h) - 1.0)
    h = _gat_layer(h, aeff, W2_ref[...], as2_ref[...], ad2_ref[...],
                   b2_ref[...], 1)
    out_ref[0] = h


def _gat3(A, xp, W0, as0v, ad0v, b0v, W1, as1v, ad1v, b1v,
          W2, as2v, ad2v, b2v):
    full = lambda shape: pl.BlockSpec(shape, lambda r: (0,) * len(shape))
    return pl.pallas_call(
        _gat3_body,
        grid=(R,),
        in_specs=[
            full((NP, NP)),
            pl.BlockSpec((1, NP, 3), lambda r: (r, 0, 0)),
            full((3, 64)), full((1, 64)), full((1, 64)), full((1, 64)),
            full((64, 64)), full((1, 64)), full((1, 64)), full((1, 64)),
            full((64, 64)), full((1, 64)), full((1, 64)), full((1, 64)),
        ],
        out_specs=pl.BlockSpec((1, NP, 64), lambda r: (r, 0, 0)),
        out_shape=jax.ShapeDtypeStruct((R, NP, 64), jnp.float32),
    )(A, xp, W0, as0v, ad0v, b0v, W1, as1v, ad1v, b1v, W2, as2v, ad2v, b2v)


NBLK = 448
NJ = NP // NBLK


def _lstm(ys, Wih, Whh, bsum):
    y_all = jnp.concatenate(ys, axis=0)
    gx = _nt(y_all, Wih) + bsum
    h = jnp.zeros((NBLK, 64), jnp.float32)
    c = jnp.zeros((NBLK, 64), jnp.float32)
    outs = []
    for t in range(T):
        g = gx[t * NBLK:(t + 1) * NBLK, :] + _nt(h, Whh)
        i = jax.nn.sigmoid(g[:, 0:64])
        f = jax.nn.sigmoid(g[:, 64:128])
        gg = jnp.tanh(g[:, 128:192])
        o = jax.nn.sigmoid(g[:, 192:256])
        c = f * c + i * gg
        h = o * jnp.tanh(c)
        outs.append(h)
    return outs, h


def _temporal_body(tf_ref, cwT_ref, cb_ref, Wih0_ref, Whh0_ref, bL0_ref,
                   Wih1_ref, Whh1_ref, bL1_ref, ow1_ref, ob1_ref,
                   ow2_ref, ob2_ref, out_ref):
    xb = tf_ref[0]
    xf = xb.reshape(T * NBLK, 64)
    y0 = _nt(xf, cwT_ref[0])
    y1 = _nt(xf, cwT_ref[1])
    y2 = _nt(xf, cwT_ref[2])
    cb = cb_ref[...]
    ys = []
    for t in range(T):
        y = y1[t * NBLK:(t + 1) * NBLK, :] + cb
        if t > 0:
            y = y + y0[(t - 1) * NBLK:t * NBLK, :]
        if t < T - 1:
            y = y + y2[(t + 1) * NBLK:(t + 2) * NBLK, :]
        ys.append(y)
    ys1, _ = _lstm(ys, Wih0_ref[...], Whh0_ref[...], bL0_ref[...])
    _, h2 = _lstm(ys1, Wih1_ref[...], Whh1_ref[...], bL1_ref[...])
    hid = jnp.maximum(_nt(h2, ow1_ref[...]) + ob1_ref[...], 0.0)
    out_ref[0] = _nt(hid, ow2_ref[...]) + ob2_ref[...]


def _temporal(tf4, cwT, cbv, Wih0, Whh0, bL0, Wih1, Whh1, bL1,
              ow1, ob1v, ow2, ob2v):
    full = lambda shape: pl.BlockSpec(shape, lambda b, j: (0,) * len(shape))
    return pl.pallas_call(
        _temporal_body,
        grid=(B, NJ),
        in_specs=[
            pl.BlockSpec((1, T, NBLK, 64), lambda b, j: (b, 0, j, 0)),
            full((3, 64, 64)), full((1, 64)),
            full((256, 64)), full((256, 64)), full((1, 256)),
            full((256, 64)), full((256, 64)), full((1, 256)),
            full((32, 64)), full((1, 32)), full((24, 32)), full((1, 24)),
        ],
        out_specs=pl.BlockSpec((1, NBLK, 24), lambda b, j: (b, j, 0)),
        out_shape=jax.ShapeDtypeStruct((B, NP, 24), jnp.float32),
    )(tf4, cwT, cbv, Wih0, Whh0, bL0, Wih1, Whh1, bL1, ow1, ob1v, ow2, ob2v)


def kernel(x, W0, as0, ad0, b0, W1, as1, ad1, b1, W2, as2, ad2, b2, cw, cb,
           Wih0, Whh0, bih0, bhh0, Wih1, Whh1, bih1, bhh1, ow1, ob1, ow2,
           ob2, edge_index):
    src = edge_index[0].astype(jnp.int32)
    dst = edge_index[1].astype(jnp.int32)
    A = _build_adj(src, dst).reshape(NP, NP)

    xp = jnp.pad(x, ((0, 0), (0, 0), (0, NP - N), (0, 0)))
    xp = xp.reshape(R, NP, 3)
    tf = _gat3(A, xp,
               W0, as0.reshape(1, 64), ad0.reshape(1, 64), b0.reshape(1, 64),
               W1, as1.reshape(1, 64), ad1.reshape(1, 64), b1.reshape(1, 64),
               W2, as2.reshape(1, 64), ad2.reshape(1, 64), b2.reshape(1, 64))
    tf4 = tf.reshape(B, T, NP, 64)

    cwT = jnp.transpose(cw, (2, 0, 1))
    y = _temporal(tf4, cwT, cb.reshape(1, 64),
                  Wih0, Whh0, (bih0 + bhh0).reshape(1, 256),
                  Wih1, Whh1, (bih1 + bhh1).reshape(1, 256),
                  ow1, ob1.reshape(1, 32), ow2, ob2.reshape(1, 24))
    return y[:, :N, :]


# speedup vs baseline: 331.9102x; 1.2100x over previous
import functools

import jax
import jax.numpy as jnp
from jax import lax
from jax.experimental import pallas as pl
from jax.experimental.pallas import tpu as pltpu
from jax.experimental.pallas import tpu_sc as plsc

N = 883
NP = 896
B, T = 4, 12
R = B * T
E = 14128
NTILES = 32
ROWS_PER_TILE = NP // NTILES
BLK_WORDS = ROWS_PER_TILE * NP
LANES = 16


def _sc_lane_gather(v, idx):
    return lax.gather(
        v, idx.reshape(LANES, 1),
        lax.GatherDimensionNumbers(
            offset_dims=(), collapsed_slice_dims=(0,), start_index_map=(0,)),
        (1,), mode=lax.GatherScatterMode.PROMISE_IN_BOUNDS)


def _build_adj_kernel(src_hbm, dst_hbm, out_hbm, src_v, dst_v, acc):
    wid = lax.axis_index("c") * 16 + lax.axis_index("s")
    base_row = wid * ROWS_PER_TILE
    iota = lax.broadcasted_iota(jnp.int32, (LANES,), 0)
    zeros16 = jnp.zeros((LANES,), jnp.float32)
    ones16 = jnp.ones((LANES,), jnp.float32)

    pltpu.sync_copy(src_hbm, src_v)
    pltpu.sync_copy(dst_hbm, dst_v)

    def _zero(j, carry):
        acc[pl.ds(j * LANES, LANES)] = zeros16
        return carry
    lax.fori_loop(0, BLK_WORDS // LANES, _zero, 0)

    big = jnp.int32(0x40000000)

    def _edges(g, carry):
        s = src_v[pl.ds(g * LANES, LANES)]
        d = dst_v[pl.ds(g * LANES, LANES)]
        rel = d - base_row
        inr = (rel >= 0) & (rel < ROWS_PER_TILE)
        idx = rel * NP + s
        key = jnp.where(inr, idx, big + iota)
        cnt, last = plsc.scan_count(key, mask=inr)
        wmask = last & inr
        widx = jnp.where(wmask, key, 0)
        plsc.addupdate_scatter(acc, [widx], cnt.astype(jnp.float32),
                               mask=wmask)
        return carry
    lax.fori_loop(0, E // LANES, _edges, 0)

    diag0 = iota * (NP + 1) + base_row
    plsc.addupdate_scatter(acc, [diag0], ones16,
                           mask=iota < jnp.int32(ROWS_PER_TILE))
    diag1 = (iota + LANES) * (NP + 1) + base_row
    plsc.addupdate_scatter(acc, [jnp.where(iota < ROWS_PER_TILE - LANES,
                                           diag1, 0)],
                           ones16,
                           mask=iota < jnp.int32(ROWS_PER_TILE - LANES))

    pltpu.sync_copy(acc, out_hbm.at[pl.ds(wid * BLK_WORDS, BLK_WORDS)])


def _build_adj(src, dst):
    k = functools.partial(
        pl.kernel,
        out_type=jax.ShapeDtypeStruct((NP * NP,), jnp.float32),
        mesh=plsc.VectorSubcoreMesh(core_axis_name="c", subcore_axis_name="s"),
        compiler_params=pltpu.CompilerParams(needs_layout_passes=False),
        scratch_types=[
            pltpu.VMEM((E,), jnp.int32),
            pltpu.VMEM((E,), jnp.int32),
            pltpu.VMEM((BLK_WORDS,), jnp.float32),
        ],
    )(_build_adj_kernel)
    return k(src, dst)


def _nt(a, b):
    return lax.dot_general(a, b, (((1,), (1,)), ((), ())),
                           preferred_element_type=jnp.float32)


def _gat_layer(h_in, aeff, W, asv, adv, bias, heads):
    hm = jnp.dot(h_in, W, preferred_element_type=jnp.float32)
    hs = hm * asv
    hd = hm * adv
    fo = 64 // heads
    if heads > 1:
        gt = (lax.broadcasted_iota(jnp.int32, (heads, 64), 1) // fo
              == lax.broadcasted_iota(jnp.int32, (heads, 64), 0))
        gt = gt.astype(jnp.float32)
        al_st = _nt(gt, hs)
        al_d = _nt(hd, gt)
    else:
        ones8 = jnp.ones((8, 64), jnp.float32)
        al_st = _nt(ones8, hs)[0:1, :]
        al_d = _nt(hd, ones8)[:, 0:1]
    zcol = jnp.zeros((W.shape[0], 1), jnp.float32)
    e_last = (lax.broadcasted_iota(jnp.int32, (1, fo + 1), 1)
              == fo).astype(jnp.float32)
    outs = []
    for hh in range(heads):
        avs = al_st[hh:hh + 1, :]
        amax = jnp.max(avs, axis=1, keepdims=True)
        zd = al_d[:, hh:hh + 1] + amax
        mp = jnp.maximum(zd, 0.2 * zd)
        u1 = jnp.exp(zd - mp).astype(jnp.bfloat16)
        u2 = jnp.exp(0.2 * zd - mp).astype(jnp.bfloat16)
        vs = avs - amax
        v1 = jnp.exp(vs).astype(jnp.bfloat16)
        v2 = jnp.exp(0.2 * vs).astype(jnp.bfloat16)
        p = jnp.maximum(u1 * v1, u2 * v2) * aeff
        wcat = jnp.concatenate([W[:, hh * fo:(hh + 1) * fo], zcol], axis=1)
        hcat = (jnp.dot(h_in, wcat, preferred_element_type=jnp.float32)
                + e_last).astype(jnp.bfloat16)
        phd = jnp.dot(p, hcat, preferred_element_type=jnp.float32)
        outs.append(phd[:, :fo] / (phd[:, fo:fo + 1] + 1e-16))
    out = outs[0] if heads == 1 else jnp.concatenate(outs, axis=1)
    return out + bias


def _gat3_body(A_ref, x_ref, W0_ref, as0_ref, ad0_ref, b0_ref,
               W1_ref, as1_ref, ad1_ref, b1_ref,
               W2_ref, as2_ref, ad2_ref, b2_ref, out_ref):
    aeff = A_ref[...]
    x = x_ref[0]
    h = _gat_layer(x, aeff, W0_ref[...], as0_ref[...], ad0_ref[...],
                   b0_ref[...], 8)
    h = jnp.where(h > 0.0, h, jnp.exp(h) - 1.0)
    h = _gat_layer(h, aeff, W1_ref[...], as1_ref[...], ad1_ref[...],
                   b1_ref[...], 8)
    h = jnp.where(h > 0.0, h, jnp.exp(h) - 1.0)
    h = _gat_layer(h, aeff, W2_ref[...], as2_ref[...], ad2_ref[...],
                   b2_ref[...], 1)
    out_ref[0] = h


def _gat3(A, xp, W0, as0v, ad0v, b0v, W1, as1v, ad1v, b1v,
          W2, as2v, ad2v, b2v):
    full = lambda shape: pl.BlockSpec(shape, lambda r: (0,) * len(shape))
    return pl.pallas_call(
        _gat3_body,
        grid=(R,),
        in_specs=[
            full((NP, NP)),
            pl.BlockSpec((1, NP, 3), lambda r: (r, 0, 0)),
            full((3, 64)), full((1, 64)), full((1, 64)), full((1, 64)),
            full((64, 64)), full((1, 64)), full((1, 64)), full((1, 64)),
            full((64, 64)), full((1, 64)), full((1, 64)), full((1, 64)),
        ],
        out_specs=pl.BlockSpec((1, NP, 64), lambda r: (r, 0, 0)),
        out_shape=jax.ShapeDtypeStruct((R, NP, 64), jnp.float32),
    )(A, xp, W0, as0v, ad0v, b0v, W1, as1v, ad1v, b1v, W2, as2v, ad2v, b2v)


NBLK = 448
NJ = NP // NBLK


def _lstm(ys, Wih, Whh, bsum):
    y_all = jnp.concatenate(ys, axis=0)
    gx = _nt(y_all, Wih) + bsum
    h = jnp.zeros((NBLK, 64), jnp.float32)
    c = jnp.zeros((NBLK, 64), jnp.float32)
    outs = []
    for t in range(T):
        g = gx[t * NBLK:(t + 1) * NBLK, :] + _nt(h, Whh)
        i = jax.nn.sigmoid(g[:, 0:64])
        f = jax.nn.sigmoid(g[:, 64:128])
        gg = jnp.tanh(g[:, 128:192])
        o = jax.nn.sigmoid(g[:, 192:256])
        c = f * c + i * gg
        h = o * jnp.tanh(c)
        outs.append(h)
    return outs, h


def _temporal_body(tf_ref, cwT_ref, cb_ref, Wih0_ref, Whh0_ref, bL0_ref,
                   Wih1_ref, Whh1_ref, bL1_ref, ow1_ref, ob1_ref,
                   ow2_ref, ob2_ref, out_ref):
    xb = tf_ref[0]
    xf = xb.reshape(T * NBLK, 64)
    y0 = _nt(xf, cwT_ref[0])
    y1 = _nt(xf, cwT_ref[1])
    y2 = _nt(xf, cwT_ref[2])
    cb = cb_ref[...]
    ys = []
    for t in range(T):
        y = y1[t * NBLK:(t + 1) * NBLK, :] + cb
        if t > 0:
            y = y + y0[(t - 1) * NBLK:t * NBLK, :]
        if t < T - 1:
            y = y + y2[(t + 1) * NBLK:(t + 2) * NBLK, :]
        ys.append(y)
    ys1, _ = _lstm(ys, Wih0_ref[...], Whh0_ref[...], bL0_ref[...])
    _, h2 = _lstm(ys1, Wih1_ref[...], Whh1_ref[...], bL1_ref[...])
    hid = jnp.maximum(_nt(h2, ow1_ref[...]) + ob1_ref[...], 0.0)
    out_ref[0] = _nt(hid, ow2_ref[...]) + ob2_ref[...]


def _temporal(tf4, cwT, cbv, Wih0, Whh0, bL0, Wih1, Whh1, bL1,
              ow1, ob1v, ow2, ob2v):
    full = lambda shape: pl.BlockSpec(shape, lambda b, j: (0,) * len(shape))
    return pl.pallas_call(
        _temporal_body,
        grid=(B, NJ),
        in_specs=[
            pl.BlockSpec((1, T, NBLK, 64), lambda b, j: (b, 0, j, 0)),
            full((3, 64, 64)), full((1, 64)),
            full((256, 64)), full((256, 64)), full((1, 256)),
            full((256, 64)), full((256, 64)), full((1, 256)),
            full((32, 64)), full((1, 32)), full((24, 32)), full((1, 24)),
        ],
        out_specs=pl.BlockSpec((1, NBLK, 24), lambda b, j: (b, j, 0)),
        out_shape=jax.ShapeDtypeStruct((B, NP, 24), jnp.float32),
    )(tf4, cwT, cbv, Wih0, Whh0, bL0, Wih1, Whh1, bL1, ow1, ob1v, ow2, ob2v)


def kernel(x, W0, as0, ad0, b0, W1, as1, ad1, b1, W2, as2, ad2, b2, cw, cb,
           Wih0, Whh0, bih0, bhh0, Wih1, Whh1, bih1, bhh1, ow1, ob1, ow2,
           ob2, edge_index):
    src = edge_index[0].astype(jnp.int32)
    dst = edge_index[1].astype(jnp.int32)
    A = _build_adj(src, dst).reshape(NP, NP).astype(jnp.bfloat16)

    xp = jnp.pad(x, ((0, 0), (0, 0), (0, NP - N), (0, 0)))
    xp = xp.reshape(R, NP, 3)
    tf = _gat3(A, xp,
               W0, as0.reshape(1, 64), ad0.reshape(1, 64), b0.reshape(1, 64),
               W1, as1.reshape(1, 64), ad1.reshape(1, 64), b1.reshape(1, 64),
               W2, as2.reshape(1, 64), ad2.reshape(1, 64), b2.reshape(1, 64))
    tf4 = tf.reshape(B, T, NP, 64)

    cwT = jnp.transpose(cw, (2, 0, 1))
    y = _temporal(tf4, cwT, cb.reshape(1, 64),
                  Wih0, Whh0, (bih0 + bhh0).reshape(1, 256),
                  Wih1, Whh1, (bih1 + bhh1).reshape(1, 256),
                  ow1, ob1.reshape(1, 32), ow2, ob2.reshape(1, 24))
    return y[:, :N, :]


# speedup vs baseline: 380.4678x; 1.1463x over previous
import functools

import jax
import jax.numpy as jnp
from jax import lax
from jax.experimental import pallas as pl
from jax.experimental.pallas import tpu as pltpu
from jax.experimental.pallas import tpu_sc as plsc

N = 883
NP = 896
B, T = 4, 12
R = B * T
E = 14128
NTILES = 32
ROWS_PER_TILE = NP // NTILES
BLK_WORDS = ROWS_PER_TILE * NP
LANES = 16


def _sc_lane_gather(v, idx):
    return lax.gather(
        v, idx.reshape(LANES, 1),
        lax.GatherDimensionNumbers(
            offset_dims=(), collapsed_slice_dims=(0,), start_index_map=(0,)),
        (1,), mode=lax.GatherScatterMode.PROMISE_IN_BOUNDS)


def _build_adj_kernel(src_hbm, dst_hbm, out_hbm, src_v, dst_v, acc):
    wid = lax.axis_index("c") * 16 + lax.axis_index("s")
    base_row = wid * ROWS_PER_TILE
    iota = lax.broadcasted_iota(jnp.int32, (LANES,), 0)
    zeros16 = jnp.zeros((LANES,), jnp.float32)
    ones16 = jnp.ones((LANES,), jnp.float32)

    pltpu.sync_copy(src_hbm, src_v)
    pltpu.sync_copy(dst_hbm, dst_v)

    def _zero(j, carry):
        acc[pl.ds(j * LANES, LANES)] = zeros16
        return carry
    lax.fori_loop(0, BLK_WORDS // LANES, _zero, 0)

    big = jnp.int32(0x40000000)

    def _edges(g, carry):
        s = src_v[pl.ds(g * LANES, LANES)]
        d = dst_v[pl.ds(g * LANES, LANES)]
        rel = d - base_row
        inr = (rel >= 0) & (rel < ROWS_PER_TILE)
        idx = rel * NP + s
        key = jnp.where(inr, idx, big + iota)
        cnt, last = plsc.scan_count(key, mask=inr)
        wmask = last & inr
        widx = jnp.where(wmask, key, 0)
        plsc.addupdate_scatter(acc, [widx], cnt.astype(jnp.float32),
                               mask=wmask)
        return carry
    lax.fori_loop(0, E // LANES, _edges, 0)

    diag0 = iota * (NP + 1) + base_row
    plsc.addupdate_scatter(acc, [diag0], ones16,
                           mask=iota < jnp.int32(ROWS_PER_TILE))
    diag1 = (iota + LANES) * (NP + 1) + base_row
    plsc.addupdate_scatter(acc, [jnp.where(iota < ROWS_PER_TILE - LANES,
                                           diag1, 0)],
                           ones16,
                           mask=iota < jnp.int32(ROWS_PER_TILE - LANES))

    pltpu.sync_copy(acc, out_hbm.at[pl.ds(wid * BLK_WORDS, BLK_WORDS)])


def _build_adj(src, dst):
    k = functools.partial(
        pl.kernel,
        out_type=jax.ShapeDtypeStruct((NP * NP,), jnp.float32),
        mesh=plsc.VectorSubcoreMesh(core_axis_name="c", subcore_axis_name="s"),
        compiler_params=pltpu.CompilerParams(needs_layout_passes=False),
        scratch_types=[
            pltpu.VMEM((E,), jnp.int32),
            pltpu.VMEM((E,), jnp.int32),
            pltpu.VMEM((BLK_WORDS,), jnp.float32),
        ],
    )(_build_adj_kernel)
    return k(src, dst)


def _nt(a, b):
    return lax.dot_general(a, b, (((1,), (1,)), ((), ())),
                           preferred_element_type=jnp.float32)


def _gat_layer(h_in, aeff, W, wcat, asv, adv, bias, heads):
    hm = jnp.dot(h_in, W, preferred_element_type=jnp.float32)
    hs = hm * asv
    hd = hm * adv
    fo = 64 // heads
    if heads > 1:
        gt = (lax.broadcasted_iota(jnp.int32, (heads, 64), 1) // fo
              == lax.broadcasted_iota(jnp.int32, (heads, 64), 0))
        gt = gt.astype(jnp.float32)
        al_st = _nt(gt, hs)
        al_d = _nt(hd, gt)
    else:
        ones8 = jnp.ones((8, 64), jnp.float32)
        al_st = _nt(ones8, hs)[0:1, :]
        al_d = _nt(hd, ones8)[:, 0:1]
    amax_r = jnp.max(al_st, axis=1, keepdims=True)
    zd = al_d + amax_r.reshape(1, heads)
    mp = jnp.maximum(zd, 0.2 * zd)
    u1a = jnp.exp(zd - mp).astype(jnp.bfloat16)
    u2a = jnp.exp(0.2 * zd - mp).astype(jnp.bfloat16)
    vs = al_st - amax_r
    v1a = jnp.exp(vs).astype(jnp.bfloat16)
    v2a = jnp.exp(0.2 * vs).astype(jnp.bfloat16)
    bw = wcat.shape[1] // heads
    e_ones = (lax.broadcasted_iota(jnp.int32, (1, heads * bw), 1) % bw
              == fo).astype(jnp.float32)
    hcat = (jnp.dot(h_in, wcat, preferred_element_type=jnp.float32)
            + e_ones).astype(jnp.bfloat16)
    outs = []
    for hh in range(heads):
        u1 = u1a[:, hh:hh + 1]
        u2 = u2a[:, hh:hh + 1]
        v1 = v1a[hh:hh + 1, :]
        v2 = v2a[hh:hh + 1, :]
        p = jnp.maximum(u1 * v1, u2 * v2) * aeff
        phd = jnp.dot(p, hcat, preferred_element_type=jnp.float32)
        outs.append(phd[:, hh * bw:hh * bw + fo]
                    / (phd[:, hh * bw + fo:hh * bw + fo + 1] + 1e-16))
    out = outs[0] if heads == 1 else jnp.concatenate(outs, axis=1)
    return out + bias


def _gat3_body(A_ref, x_ref, W0_ref, wc0_ref, as0_ref, ad0_ref, b0_ref,
               W1_ref, wc1_ref, as1_ref, ad1_ref, b1_ref,
               W2_ref, wc2_ref, as2_ref, ad2_ref, b2_ref, out_ref):
    aeff = A_ref[...]
    x = x_ref[0]
    h = _gat_layer(x, aeff, W0_ref[...], wc0_ref[...], as0_ref[...],
                   ad0_ref[...], b0_ref[...], 8)
    h = jnp.where(h > 0.0, h, jnp.exp(h) - 1.0)
    h = _gat_layer(h, aeff, W1_ref[...], wc1_ref[...], as1_ref[...],
                   ad1_ref[...], b1_ref[...], 8)
    h = jnp.where(h > 0.0, h, jnp.exp(h) - 1.0)
    h = _gat_layer(h, aeff, W2_ref[...], wc2_ref[...], as2_ref[...],
                   ad2_ref[...], b2_ref[...], 1)
    out_ref[0] = h


def _make_wcat(W, heads):
    fo = 64 // heads
    bw = 16 if heads > 1 else 128
    z = jnp.zeros((W.shape[0], bw - fo), W.dtype)
    return jnp.concatenate(
        [jnp.concatenate([W[:, hh * fo:(hh + 1) * fo], z], axis=1)
         for hh in range(heads)], axis=1)


def _gat3(A, xp, W0, wc0, as0v, ad0v, b0v, W1, wc1, as1v, ad1v, b1v,
          W2, wc2, as2v, ad2v, b2v):
    full = lambda shape: pl.BlockSpec(shape, lambda r: (0,) * len(shape))
    return pl.pallas_call(
        _gat3_body,
        grid=(R,),
        in_specs=[
            full((NP, NP)),
            pl.BlockSpec((1, NP, 3), lambda r: (r, 0, 0)),
            full((3, 64)), full((3, 128)),
            full((1, 64)), full((1, 64)), full((1, 64)),
            full((64, 64)), full((64, 128)),
            full((1, 64)), full((1, 64)), full((1, 64)),
            full((64, 64)), full((64, 128)),
            full((1, 64)), full((1, 64)), full((1, 64)),
        ],
        out_specs=pl.BlockSpec((1, NP, 64), lambda r: (r, 0, 0)),
        out_shape=jax.ShapeDtypeStruct((R, NP, 64), jnp.float32),
    )(A, xp, W0, wc0, as0v, ad0v, b0v, W1, wc1, as1v, ad1v, b1v,
      W2, wc2, as2v, ad2v, b2v)


NBLK = 448
NJ = NP // NBLK


def _lstm(ys, Wih, Whh, bsum):
    y_all = jnp.concatenate(ys, axis=0)
    gx = _nt(y_all, Wih) + bsum
    h = jnp.zeros((NBLK, 64), jnp.float32)
    c = jnp.zeros((NBLK, 64), jnp.float32)
    outs = []
    for t in range(T):
        g = gx[t * NBLK:(t + 1) * NBLK, :] + _nt(h, Whh)
        i = jax.nn.sigmoid(g[:, 0:64])
        f = jax.nn.sigmoid(g[:, 64:128])
        gg = jnp.tanh(g[:, 128:192])
        o = jax.nn.sigmoid(g[:, 192:256])
        c = f * c + i * gg
        h = o * jnp.tanh(c)
        outs.append(h)
    return outs, h


def _temporal_body(tf_ref, cwT_ref, cb_ref, Wih0_ref, Whh0_ref, bL0_ref,
                   Wih1_ref, Whh1_ref, bL1_ref, ow1_ref, ob1_ref,
                   ow2_ref, ob2_ref, out_ref):
    xb = tf_ref[0]
    xf = xb.reshape(T * NBLK, 64)
    y0 = _nt(xf, cwT_ref[0])
    y1 = _nt(xf, cwT_ref[1])
    y2 = _nt(xf, cwT_ref[2])
    cb = cb_ref[...]
    ys = []
    for t in range(T):
        y = y1[t * NBLK:(t + 1) * NBLK, :] + cb
        if t > 0:
            y = y + y0[(t - 1) * NBLK:t * NBLK, :]
        if t < T - 1:
            y = y + y2[(t + 1) * NBLK:(t + 2) * NBLK, :]
        ys.append(y)
    ys1, _ = _lstm(ys, Wih0_ref[...], Whh0_ref[...], bL0_ref[...])
    _, h2 = _lstm(ys1, Wih1_ref[...], Whh1_ref[...], bL1_ref[...])
    hid = jnp.maximum(_nt(h2, ow1_ref[...]) + ob1_ref[...], 0.0)
    out_ref[0] = _nt(hid, ow2_ref[...]) + ob2_ref[...]


def _temporal(tf4, cwT, cbv, Wih0, Whh0, bL0, Wih1, Whh1, bL1,
              ow1, ob1v, ow2, ob2v):
    full = lambda shape: pl.BlockSpec(shape, lambda b, j: (0,) * len(shape))
    return pl.pallas_call(
        _temporal_body,
        grid=(B, NJ),
        in_specs=[
            pl.BlockSpec((1, T, NBLK, 64), lambda b, j: (b, 0, j, 0)),
            full((3, 64, 64)), full((1, 64)),
            full((256, 64)), full((256, 64)), full((1, 256)),
            full((256, 64)), full((256, 64)), full((1, 256)),
            full((32, 64)), full((1, 32)), full((24, 32)), full((1, 24)),
        ],
        out_specs=pl.BlockSpec((1, NBLK, 24), lambda b, j: (b, j, 0)),
        out_shape=jax.ShapeDtypeStruct((B, NP, 24), jnp.float32),
    )(tf4, cwT, cbv, Wih0, Whh0, bL0, Wih1, Whh1, bL1, ow1, ob1v, ow2, ob2v)


def kernel(x, W0, as0, ad0, b0, W1, as1, ad1, b1, W2, as2, ad2, b2, cw, cb,
           Wih0, Whh0, bih0, bhh0, Wih1, Whh1, bih1, bhh1, ow1, ob1, ow2,
           ob2, edge_index):
    src = edge_index[0].astype(jnp.int32)
    dst = edge_index[1].astype(jnp.int32)
    A = _build_adj(src, dst).reshape(NP, NP).astype(jnp.bfloat16)

    xp = jnp.pad(x, ((0, 0), (0, 0), (0, NP - N), (0, 0)))
    xp = xp.reshape(R, NP, 3)
    tf = _gat3(A, xp,
               W0, _make_wcat(W0, 8), as0.reshape(1, 64),
               ad0.reshape(1, 64), b0.reshape(1, 64),
               W1, _make_wcat(W1, 8), as1.reshape(1, 64),
               ad1.reshape(1, 64), b1.reshape(1, 64),
               W2, _make_wcat(W2, 1), as2.reshape(1, 64),
               ad2.reshape(1, 64), b2.reshape(1, 64))
    tf4 = tf.reshape(B, T, NP, 64)

    cwT = jnp.transpose(cw, (2, 0, 1))
    y = _temporal(tf4, cwT, cb.reshape(1, 64),
                  Wih0, Whh0, (bih0 + bhh0).reshape(1, 256),
                  Wih1, Whh1, (bih1 + bhh1).reshape(1, 256),
                  ow1, ob1.reshape(1, 32), ow2, ob2.reshape(1, 24))
    return y[:, :N, :]


# speedup vs baseline: 390.4077x; 1.0261x over previous
import functools

import jax
import jax.numpy as jnp
from jax import lax
from jax.experimental import pallas as pl
from jax.experimental.pallas import tpu as pltpu
from jax.experimental.pallas import tpu_sc as plsc

N = 883
NP = 896
B, T = 4, 12
R = B * T
E = 14128
NTILES = 32
ROWS_PER_TILE = NP // NTILES
BLK_WORDS = ROWS_PER_TILE * NP
LANES = 16


def _sc_lane_gather(v, idx):
    return lax.gather(
        v, idx.reshape(LANES, 1),
        lax.GatherDimensionNumbers(
            offset_dims=(), collapsed_slice_dims=(0,), start_index_map=(0,)),
        (1,), mode=lax.GatherScatterMode.PROMISE_IN_BOUNDS)


def _build_adj_kernel(src_hbm, dst_hbm, out_hbm, src_v, dst_v, acc):
    wid = lax.axis_index("c") * 16 + lax.axis_index("s")
    base_row = wid * ROWS_PER_TILE
    iota = lax.broadcasted_iota(jnp.int32, (LANES,), 0)
    zeros16 = jnp.zeros((LANES,), jnp.float32)
    ones16 = jnp.ones((LANES,), jnp.float32)

    pltpu.sync_copy(src_hbm, src_v)
    pltpu.sync_copy(dst_hbm, dst_v)

    def _zero(j, carry):
        acc[pl.ds(j * LANES, LANES)] = zeros16
        return carry
    lax.fori_loop(0, BLK_WORDS // LANES, _zero, 0)

    big = jnp.int32(0x40000000)

    def _edges(g, carry):
        s = src_v[pl.ds(g * LANES, LANES)]
        d = dst_v[pl.ds(g * LANES, LANES)]
        rel = d - base_row
        inr = (rel >= 0) & (rel < ROWS_PER_TILE)
        idx = rel * NP + s
        key = jnp.where(inr, idx, big + iota)
        cnt, last = plsc.scan_count(key, mask=inr)
        wmask = last & inr
        widx = jnp.where(wmask, key, 0)
        plsc.addupdate_scatter(acc, [widx], cnt.astype(jnp.float32),
                               mask=wmask)
        return carry
    lax.fori_loop(0, E // LANES, _edges, 0)

    diag0 = iota * (NP + 1) + base_row
    plsc.addupdate_scatter(acc, [diag0], ones16,
                           mask=iota < jnp.int32(ROWS_PER_TILE))
    diag1 = (iota + LANES) * (NP + 1) + base_row
    plsc.addupdate_scatter(acc, [jnp.where(iota < ROWS_PER_TILE - LANES,
                                           diag1, 0)],
                           ones16,
                           mask=iota < jnp.int32(ROWS_PER_TILE - LANES))

    pltpu.sync_copy(acc, out_hbm.at[pl.ds(wid * BLK_WORDS, BLK_WORDS)])


def _build_adj(src, dst):
    k = functools.partial(
        pl.kernel,
        out_type=jax.ShapeDtypeStruct((NP * NP,), jnp.float32),
        mesh=plsc.VectorSubcoreMesh(core_axis_name="c", subcore_axis_name="s"),
        compiler_params=pltpu.CompilerParams(needs_layout_passes=False),
        scratch_types=[
            pltpu.VMEM((E,), jnp.int32),
            pltpu.VMEM((E,), jnp.int32),
            pltpu.VMEM((BLK_WORDS,), jnp.float32),
        ],
    )(_build_adj_kernel)
    return k(src, dst)


def _nt(a, b):
    return lax.dot_general(a, b, (((1,), (1,)), ((), ())),
                           preferred_element_type=jnp.float32)


def _gat_layer(h_in, aeff, W, wcat, asv, adv, bias, heads):
    hm = jnp.dot(h_in, W, preferred_element_type=jnp.float32)
    hs = hm * asv
    hd = hm * adv
    fo = 64 // heads
    if heads > 1:
        gt = (lax.broadcasted_iota(jnp.int32, (heads, 64), 1) // fo
              == lax.broadcasted_iota(jnp.int32, (heads, 64), 0))
        gt = gt.astype(jnp.float32)
        al_st = _nt(gt, hs)
        al_d = _nt(hd, gt)
    else:
        ones8 = jnp.ones((8, 64), jnp.float32)
        al_st = _nt(ones8, hs)[0:1, :]
        al_d = _nt(hd, ones8)[:, 0:1]
    amax_r = jnp.max(al_st, axis=1, keepdims=True)
    zd = al_d + amax_r.reshape(1, heads)
    mp = jnp.maximum(zd, 0.2 * zd)
    u1a = jnp.exp(zd - mp).astype(jnp.bfloat16)
    u2a = jnp.exp(0.2 * zd - mp).astype(jnp.bfloat16)
    vs = al_st - amax_r
    v1a = jnp.exp(vs).astype(jnp.bfloat16)
    v2a = jnp.exp(0.2 * vs).astype(jnp.bfloat16)
    bw = wcat.shape[1] // heads
    e_ones = (lax.broadcasted_iota(jnp.int32, (1, heads * bw), 1) % bw
              == fo).astype(jnp.float32)
    hcat = (jnp.dot(h_in, wcat, preferred_element_type=jnp.float32)
            + e_ones).astype(jnp.bfloat16)
    outs = []
    for hh in range(heads):
        u1 = u1a[:, hh:hh + 1]
        u2 = u2a[:, hh:hh + 1]
        v1 = v1a[hh:hh + 1, :]
        v2 = v2a[hh:hh + 1, :]
        p = jnp.maximum(u1 * v1, u2 * v2) * aeff
        phd = jnp.dot(p, hcat, preferred_element_type=jnp.float32)
        outs.append(phd[:, hh * bw:hh * bw + fo]
                    / (phd[:, hh * bw + fo:hh * bw + fo + 1] + 1e-16))
    out = outs[0] if heads == 1 else jnp.concatenate(outs, axis=1)
    return out + bias


def _gat3_body(A_ref, x_ref, W0_ref, wc0_ref, as0_ref, ad0_ref, b0_ref,
               W1_ref, wc1_ref, as1_ref, ad1_ref, b1_ref,
               W2_ref, wc2_ref, as2_ref, ad2_ref, b2_ref, out_ref):
    aeff = A_ref[...]
    x = x_ref[0]
    h = _gat_layer(x, aeff, W0_ref[...], wc0_ref[...], as0_ref[...],
                   ad0_ref[...], b0_ref[...], 8)
    h = jnp.where(h > 0.0, h, jnp.exp(h) - 1.0)
    h = _gat_layer(h, aeff, W1_ref[...], wc1_ref[...], as1_ref[...],
                   ad1_ref[...], b1_ref[...], 8)
    h = jnp.where(h > 0.0, h, jnp.exp(h) - 1.0)
    h = _gat_layer(h, aeff, W2_ref[...], wc2_ref[...], as2_ref[...],
                   ad2_ref[...], b2_ref[...], 1)
    out_ref[0] = h


def _make_wcat(W, heads):
    fo = 64 // heads
    bw = 16 if heads > 1 else 128
    z = jnp.zeros((W.shape[0], bw - fo), W.dtype)
    return jnp.concatenate(
        [jnp.concatenate([W[:, hh * fo:(hh + 1) * fo], z], axis=1)
         for hh in range(heads)], axis=1)


def _gat3(A, xp, W0, wc0, as0v, ad0v, b0v, W1, wc1, as1v, ad1v, b1v,
          W2, wc2, as2v, ad2v, b2v):
    full = lambda shape: pl.BlockSpec(shape, lambda r: (0,) * len(shape))
    return pl.pallas_call(
        _gat3_body,
        grid=(R,),
        in_specs=[
            full((NP, NP)),
            pl.BlockSpec((1, NP, 3), lambda r: (r, 0, 0)),
            full((3, 64)), full((3, 128)),
            full((1, 64)), full((1, 64)), full((1, 64)),
            full((64, 64)), full((64, 128)),
            full((1, 64)), full((1, 64)), full((1, 64)),
            full((64, 64)), full((64, 128)),
            full((1, 64)), full((1, 64)), full((1, 64)),
        ],
        out_specs=pl.BlockSpec((1, NP, 64), lambda r: (r, 0, 0)),
        out_shape=jax.ShapeDtypeStruct((R, NP, 64), jnp.float32),
    )(A, xp, W0, wc0, as0v, ad0v, b0v, W1, wc1, as1v, ad1v, b1v,
      W2, wc2, as2v, ad2v, b2v)


NBLK = 896
NJ = NP // NBLK


def _lstm(ys, Wih, Whh, bsum):
    y_all = jnp.concatenate(ys, axis=0)
    gx = _nt(y_all, Wih) + bsum
    h = jnp.zeros((NBLK, 64), jnp.float32)
    c = jnp.zeros((NBLK, 64), jnp.float32)
    outs = []
    for t in range(T):
        g = gx[t * NBLK:(t + 1) * NBLK, :] + _nt(h, Whh)
        i = jax.nn.sigmoid(g[:, 0:64])
        f = jax.nn.sigmoid(g[:, 64:128])
        gg = jnp.tanh(g[:, 128:192])
        o = jax.nn.sigmoid(g[:, 192:256])
        c = f * c + i * gg
        h = o * jnp.tanh(c)
        outs.append(h)
    return outs, h


def _temporal_body(tf_ref, cwT_ref, cb_ref, Wih0_ref, Whh0_ref, bL0_ref,
                   Wih1_ref, Whh1_ref, bL1_ref, ow1_ref, ob1_ref,
                   ow2_ref, ob2_ref, out_ref):
    xb = tf_ref[0]
    xf = xb.reshape(T * NBLK, 64)
    y0 = _nt(xf, cwT_ref[0])
    y1 = _nt(xf, cwT_ref[1])
    y2 = _nt(xf, cwT_ref[2])
    cb = cb_ref[...]
    ys = []
    for t in range(T):
        y = y1[t * NBLK:(t + 1) * NBLK, :] + cb
        if t > 0:
            y = y + y0[(t - 1) * NBLK:t * NBLK, :]
        if t < T - 1:
            y = y + y2[(t + 1) * NBLK:(t + 2) * NBLK, :]
        ys.append(y)
    ys1, _ = _lstm(ys, Wih0_ref[...], Whh0_ref[...], bL0_ref[...])
    _, h2 = _lstm(ys1, Wih1_ref[...], Whh1_ref[...], bL1_ref[...])
    hid = jnp.maximum(_nt(h2, ow1_ref[...]) + ob1_ref[...], 0.0)
    out_ref[0] = _nt(hid, ow2_ref[...]) + ob2_ref[...]


def _temporal(tf4, cwT, cbv, Wih0, Whh0, bL0, Wih1, Whh1, bL1,
              ow1, ob1v, ow2, ob2v):
    full = lambda shape: pl.BlockSpec(shape, lambda b, j: (0,) * len(shape))
    return pl.pallas_call(
        _temporal_body,
        grid=(B, NJ),
        in_specs=[
            pl.BlockSpec((1, T, NBLK, 64), lambda b, j: (b, 0, j, 0)),
            full((3, 64, 64)), full((1, 64)),
            full((256, 64)), full((256, 64)), full((1, 256)),
            full((256, 64)), full((256, 64)), full((1, 256)),
            full((32, 64)), full((1, 32)), full((24, 32)), full((1, 24)),
        ],
        out_specs=pl.BlockSpec((1, NBLK, 24), lambda b, j: (b, j, 0)),
        out_shape=jax.ShapeDtypeStruct((B, NP, 24), jnp.float32),
    )(tf4, cwT, cbv, Wih0, Whh0, bL0, Wih1, Whh1, bL1, ow1, ob1v, ow2, ob2v)


def kernel(x, W0, as0, ad0, b0, W1, as1, ad1, b1, W2, as2, ad2, b2, cw, cb,
           Wih0, Whh0, bih0, bhh0, Wih1, Whh1, bih1, bhh1, ow1, ob1, ow2,
           ob2, edge_index):
    src = edge_index[0].astype(jnp.int32)
    dst = edge_index[1].astype(jnp.int32)
    A = _build_adj(src, dst).reshape(NP, NP).astype(jnp.bfloat16)

    xp = jnp.pad(x, ((0, 0), (0, 0), (0, NP - N), (0, 0)))
    xp = xp.reshape(R, NP, 3)
    tf = _gat3(A, xp,
               W0, _make_wcat(W0, 8), as0.reshape(1, 64),
               ad0.reshape(1, 64), b0.reshape(1, 64),
               W1, _make_wcat(W1, 8), as1.reshape(1, 64),
               ad1.reshape(1, 64), b1.reshape(1, 64),
               W2, _make_wcat(W2, 1), as2.reshape(1, 64),
               ad2.reshape(1, 64), b2.reshape(1, 64))
    tf4 = tf.reshape(B, T, NP, 64)

    cwT = jnp.transpose(cw, (2, 0, 1))
    y = _temporal(tf4, cwT, cb.reshape(1, 64),
                  Wih0, Whh0, (bih0 + bhh0).reshape(1, 256),
                  Wih1, Whh1, (bih1 + bhh1).reshape(1, 256),
                  ow1, ob1.reshape(1, 32), ow2, ob2.reshape(1, 24))
    return y[:, :N, :]


# speedup vs baseline: 390.4133x; 1.0000x over previous
import functools

import jax
import jax.numpy as jnp
from jax import lax
from jax.experimental import pallas as pl
from jax.experimental.pallas import tpu as pltpu
from jax.experimental.pallas import tpu_sc as plsc

N = 883
NP = 896
B, T = 4, 12
R = B * T
E = 14128
NTILES = 32
ROWS_PER_TILE = NP // NTILES
BLK_WORDS = ROWS_PER_TILE * NP
LANES = 16


def _build_adj_kernel(src_hbm, dst_hbm, out_hbm, src_v, dst_v, acc):
    wid = lax.axis_index("c") * 16 + lax.axis_index("s")
    base_row = wid * ROWS_PER_TILE
    iota = lax.broadcasted_iota(jnp.int32, (LANES,), 0)
    zeros16 = jnp.zeros((LANES,), jnp.float32)
    ones16 = jnp.ones((LANES,), jnp.float32)

    pltpu.sync_copy(src_hbm, src_v)
    pltpu.sync_copy(dst_hbm, dst_v)

    def _zero(j, carry):
        acc[pl.ds(j * LANES, LANES)] = zeros16
        return carry
    lax.fori_loop(0, BLK_WORDS // LANES, _zero, 0)

    big = jnp.int32(0x40000000)

    def _edges(g, carry):
        s = src_v[pl.ds(g * LANES, LANES)]
        d = dst_v[pl.ds(g * LANES, LANES)]
        rel = d - base_row
        inr = (rel >= 0) & (rel < ROWS_PER_TILE)
        idx = rel * NP + s
        key = jnp.where(inr, idx, big + iota)
        cnt, last = plsc.scan_count(key, mask=inr)
        wmask = last & inr
        widx = jnp.where(wmask, key, 0)
        plsc.addupdate_scatter(acc, [widx], cnt.astype(jnp.float32),
                               mask=wmask)
        return carry
    lax.fori_loop(0, E // LANES, _edges, 0)

    diag0 = iota * (NP + 1) + base_row
    plsc.addupdate_scatter(acc, [diag0], ones16,
                           mask=iota < jnp.int32(ROWS_PER_TILE))
    diag1 = (iota + LANES) * (NP + 1) + base_row
    plsc.addupdate_scatter(acc, [jnp.where(iota < ROWS_PER_TILE - LANES,
                                           diag1, 0)],
                           ones16,
                           mask=iota < jnp.int32(ROWS_PER_TILE - LANES))

    pltpu.sync_copy(acc, out_hbm.at[pl.ds(wid * BLK_WORDS, BLK_WORDS)])


def _build_adj(src, dst):
    k = functools.partial(
        pl.kernel,
        out_type=jax.ShapeDtypeStruct((NP * NP,), jnp.float32),
        mesh=plsc.VectorSubcoreMesh(core_axis_name="c", subcore_axis_name="s"),
        compiler_params=pltpu.CompilerParams(needs_layout_passes=False),
        scratch_types=[
            pltpu.VMEM((E,), jnp.int32),
            pltpu.VMEM((E,), jnp.int32),
            pltpu.VMEM((BLK_WORDS,), jnp.float32),
        ],
    )(_build_adj_kernel)
    return k(src, dst)


def _nt(a, b):
    return lax.dot_general(a, b, (((1,), (1,)), ((), ())),
                           preferred_element_type=jnp.float32)


def _gat_layer(h_in, aeff, W, wcat, asv, adv, bias, heads):
    hm = jnp.dot(h_in, W, preferred_element_type=jnp.float32)
    hs = hm * asv
    hd = hm * adv
    fo = 64 // heads
    if heads > 1:
        gt = (lax.broadcasted_iota(jnp.int32, (heads, 64), 1) // fo
              == lax.broadcasted_iota(jnp.int32, (heads, 64), 0))
        gt = gt.astype(jnp.float32)
        al_st = _nt(gt, hs)
        al_d = _nt(hd, gt)
    else:
        ones8 = jnp.ones((8, 64), jnp.float32)
        al_st = _nt(ones8, hs)[0:1, :]
        al_d = _nt(hd, ones8)[:, 0:1]
    amax_r = jnp.max(al_st, axis=1, keepdims=True)
    zd = al_d + amax_r.reshape(1, heads)
    mp = jnp.maximum(zd, 0.2 * zd)
    u1a = jnp.exp(zd - mp).astype(jnp.bfloat16)
    u2a = jnp.exp(0.2 * zd - mp).astype(jnp.bfloat16)
    vs = al_st - amax_r
    v1a = jnp.exp(vs).astype(jnp.bfloat16)
    v2a = jnp.exp(0.2 * vs).astype(jnp.bfloat16)
    bw = wcat.shape[1] // heads
    e_ones = (lax.broadcasted_iota(jnp.int32, (1, heads * bw), 1) % bw
              == fo).astype(jnp.float32)
    hcat = (jnp.dot(h_in, wcat, preferred_element_type=jnp.float32)
            + e_ones).astype(jnp.bfloat16)
    outs = []
    for hh in range(heads):
        u1 = u1a[:, hh:hh + 1]
        u2 = u2a[:, hh:hh + 1]
        v1 = v1a[hh:hh + 1, :]
        v2 = v2a[hh:hh + 1, :]
        p = jnp.maximum(u1 * v1, u2 * v2) * aeff
        phd = jnp.dot(p, hcat, preferred_element_type=jnp.float32)
        outs.append(phd[:, hh * bw:hh * bw + fo]
                    / (phd[:, hh * bw + fo:hh * bw + fo + 1] + 1e-16))
    out = outs[0] if heads == 1 else jnp.concatenate(outs, axis=1)
    return out + bias


def _gat3_body(A_ref, x_ref, W0_ref, wc0_ref, as0_ref, ad0_ref, b0_ref,
               W1_ref, wc1_ref, as1_ref, ad1_ref, b1_ref,
               W2_ref, wc2_ref, as2_ref, ad2_ref, b2_ref, out_ref):
    aeff = A_ref[...]
    x = x_ref[0]
    h = _gat_layer(x, aeff, W0_ref[...], wc0_ref[...], as0_ref[...],
                   ad0_ref[...], b0_ref[...], 8)
    h = jnp.where(h > 0.0, h, jnp.exp(h) - 1.0)
    h = _gat_layer(h, aeff, W1_ref[...], wc1_ref[...], as1_ref[...],
                   ad1_ref[...], b1_ref[...], 8)
    h = jnp.where(h > 0.0, h, jnp.exp(h) - 1.0)
    h = _gat_layer(h, aeff, W2_ref[...], wc2_ref[...], as2_ref[...],
                   ad2_ref[...], b2_ref[...], 1)
    out_ref[0] = h


def _make_wcat(W, heads):
    fo = 64 // heads
    bw = 16 if heads > 1 else 128
    z = jnp.zeros((W.shape[0], bw - fo), W.dtype)
    return jnp.concatenate(
        [jnp.concatenate([W[:, hh * fo:(hh + 1) * fo], z], axis=1)
         for hh in range(heads)], axis=1)


def _gat3(A, xp, W0, wc0, as0v, ad0v, b0v, W1, wc1, as1v, ad1v, b1v,
          W2, wc2, as2v, ad2v, b2v):
    full = lambda shape: pl.BlockSpec(shape, lambda r: (0,) * len(shape))
    return pl.pallas_call(
        _gat3_body,
        grid=(R,),
        in_specs=[
            full((NP, NP)),
            pl.BlockSpec((1, NP, 3), lambda r: (r, 0, 0)),
            full((3, 64)), full((3, 128)),
            full((1, 64)), full((1, 64)), full((1, 64)),
            full((64, 64)), full((64, 128)),
            full((1, 64)), full((1, 64)), full((1, 64)),
            full((64, 64)), full((64, 128)),
            full((1, 64)), full((1, 64)), full((1, 64)),
        ],
        out_specs=pl.BlockSpec((1, NP, 64), lambda r: (r, 0, 0)),
        out_shape=jax.ShapeDtypeStruct((R, NP, 64), jnp.float32),
    )(A, xp, W0, wc0, as0v, ad0v, b0v, W1, wc1, as1v, ad1v, b1v,
      W2, wc2, as2v, ad2v, b2v)


NBLK = 896
NJ = NP // NBLK


def _lstm(ys, Wih, Whh, bsum):
    y_all = jnp.concatenate(ys, axis=0)
    gx = _nt(y_all, Wih) + bsum
    h = jnp.zeros((NBLK, 64), jnp.float32)
    c = jnp.zeros((NBLK, 64), jnp.float32)
    outs = []
    for t in range(T):
        g = gx[t * NBLK:(t + 1) * NBLK, :] + _nt(h, Whh)
        i = jax.nn.sigmoid(g[:, 0:64])
        f = jax.nn.sigmoid(g[:, 64:128])
        gg = jnp.tanh(g[:, 128:192])
        o = jax.nn.sigmoid(g[:, 192:256])
        c = f * c + i * gg
        h = o * jnp.tanh(c)
        outs.append(h)
    return outs, h


def _temporal_body(tf_ref, cwT_ref, cb_ref, Wih0_ref, Whh0_ref, bL0_ref,
                   Wih1_ref, Whh1_ref, bL1_ref, ow1_ref, ob1_ref,
                   ow2_ref, ob2_ref, out_ref):
    xb = tf_ref[0]
    xf = xb.reshape(T * NBLK, 64)
    y0 = _nt(xf, cwT_ref[0])
    y1 = _nt(xf, cwT_ref[1])
    y2 = _nt(xf, cwT_ref[2])
    cb = cb_ref[...]
    ys = []
    for t in range(T):
        y = y1[t * NBLK:(t + 1) * NBLK, :] + cb
        if t > 0:
            y = y + y0[(t - 1) * NBLK:t * NBLK, :]
        if t < T - 1:
            y = y + y2[(t + 1) * NBLK:(t + 2) * NBLK, :]
        ys.append(y)
    ys1, _ = _lstm(ys, Wih0_ref[...], Whh0_ref[...], bL0_ref[...])
    _, h2 = _lstm(ys1, Wih1_ref[...], Whh1_ref[...], bL1_ref[...])
    hid = jnp.maximum(_nt(h2, ow1_ref[...]) + ob1_ref[...], 0.0)
    out_ref[0] = _nt(hid, ow2_ref[...]) + ob2_ref[...]


def _temporal(tf4, cwT, cbv, Wih0, Whh0, bL0, Wih1, Whh1, bL1,
              ow1, ob1v, ow2, ob2v):
    full = lambda shape: pl.BlockSpec(shape, lambda b, j: (0,) * len(shape))
    return pl.pallas_call(
        _temporal_body,
        grid=(B, NJ),
        in_specs=[
            pl.BlockSpec((1, T, NBLK, 64), lambda b, j: (b, 0, j, 0)),
            full((3, 64, 64)), full((1, 64)),
            full((256, 64)), full((256, 64)), full((1, 256)),
            full((256, 64)), full((256, 64)), full((1, 256)),
            full((32, 64)), full((1, 32)), full((24, 32)), full((1, 24)),
        ],
        out_specs=pl.BlockSpec((1, NBLK, 24), lambda b, j: (b, j, 0)),
        out_shape=jax.ShapeDtypeStruct((B, NP, 24), jnp.float32),
    )(tf4, cwT, cbv, Wih0, Whh0, bL0, Wih1, Whh1, bL1, ow1, ob1v, ow2, ob2v)


def kernel(x, W0, as0, ad0, b0, W1, as1, ad1, b1, W2, as2, ad2, b2, cw, cb,
           Wih0, Whh0, bih0, bhh0, Wih1, Whh1, bih1, bhh1, ow1, ob1, ow2,
           ob2, edge_index):
    src = edge_index[0].astype(jnp.int32)
    dst = edge_index[1].astype(jnp.int32)
    A = _build_adj(src, dst).reshape(NP, NP).astype(jnp.bfloat16)

    xp = jnp.pad(x, ((0, 0), (0, 0), (0, NP - N), (0, 0)))
    xp = xp.reshape(R, NP, 3)
    tf = _gat3(A, xp,
               W0, _make_wcat(W0, 8), as0.reshape(1, 64),
               ad0.reshape(1, 64), b0.reshape(1, 64),
               W1, _make_wcat(W1, 8), as1.reshape(1, 64),
               ad1.reshape(1, 64), b1.reshape(1, 64),
               W2, _make_wcat(W2, 1), as2.reshape(1, 64),
               ad2.reshape(1, 64), b2.reshape(1, 64))
    tf4 = tf.reshape(B, T, NP, 64)

    cwT = jnp.transpose(cw, (2, 0, 1))
    y = _temporal(tf4, cwT, cb.reshape(1, 64),
                  Wih0, Whh0, (bih0 + bhh0).reshape(1, 256),
                  Wih1, Whh1, (bih1 + bhh1).reshape(1, 256),
                  ow1, ob1.reshape(1, 32), ow2, ob2.reshape(1, 24))
    return y[:, :N, :]
